# baseline (speedup 1.0000x reference)
import jax
import jax.numpy as jnp
from jax import lax
from jax.experimental import pallas as pl
from jax.experimental.pallas import tpu as pltpu
from jax.experimental.pallas import tpu_sc as plsc

_N = 10000
_E = 160000
_L = 128
_DE = 16
_STEPS = 5

_NC, _NS = 2, 16
_NW = _NC * _NS
_CHUNK = 40
_CPT = 125
_EP = _NW * _CPT * _CHUNK
_ROWS = 10112
_RPT = _ROWS // _NS

_BE = 1000
_BN = 1000


def _dot(a, b):
    return jnp.dot(a, b, preferred_element_type=jnp.float32,
                   precision=lax.Precision.HIGHEST)


def _ln(z, g, b):
    mu = jnp.mean(z, axis=-1, keepdims=True)
    zc = z - mu
    var = jnp.mean(zc * zc, axis=-1, keepdims=True)
    return zc * lax.rsqrt(var + 1e-5) * g + b


def _full(shape):
    return pl.BlockSpec(shape, lambda i: (0,) * len(shape))


def _rows(bs, ncols):
    return pl.BlockSpec((bs, ncols), lambda i: (i, 0))


def _edge_enc_body(ef, w0, b0, w1, b1, w2, b2, g, b, out):
    z = jnp.maximum(_dot(ef[...], w0[...]) + b0[...], 0.0)
    z = jnp.maximum(_dot(z, w1[...]) + b1[...], 0.0)
    z = _dot(z, w2[...]) + b2[...]
    out[...] = _ln(z, g[...], b[...])


def _edge_enc(efp, w0, b0, w1, b1, w2, b2, g, b):
    return pl.pallas_call(
        _edge_enc_body,
        grid=(_EP // _BE,),
        in_specs=[_rows(_BE, _DE), _full((_DE, _L)), _full((1, _L)),
                  _full((_L, _L)), _full((1, _L)), _full((_L, _L)),
                  _full((1, _L)), _full((1, _L)), _full((1, _L))],
        out_specs=_rows(_BE, _L),
        out_shape=jax.ShapeDtypeStruct((_EP, _L), jnp.float32),
    )(efp, w0, b0, w1, b1, w2, b2, g, b)


def _node_enc_body(x, w0, b0, w1, b1, w2, b2, g, b, wa, wb,
                   h_out, ha_out, hb_out):
    z = jnp.maximum(_dot(x[...], w0[...]) + b0[...], 0.0)
    z = jnp.maximum(_dot(z, w1[...]) + b1[...], 0.0)
    z = _dot(z, w2[...]) + b2[...]
    h = _ln(z, g[...], b[...])
    h_out[...] = h
    ha_out[...] = _dot(h, wa[...])
    hb_out[...] = _dot(h, wb[...])


def _node_enc(x, w0, b0, w1, b1, w2, b2, g, b, wa, wb):
    sds = jax.ShapeDtypeStruct((_N, _L), jnp.float32)
    return pl.pallas_call(
        _node_enc_body,
        grid=(_N // _BN,),
        in_specs=[_rows(_BN, _L)] + [_full((_L, _L)), _full((1, _L))] * 3
                 + [_full((1, _L)), _full((1, _L)),
                    _full((_L, _L)), _full((_L, _L))],
        out_specs=[_rows(_BN, _L)] * 3,
        out_shape=[sds, sds, sds],
    )(x, w0, b0, w1, b1, w2, b2, g, b, wa, wb)


def _edge_step_body(gsum, e, w1c, b1, w2, b2, w3, b3, g, b,
                    enew_out, eout_out):
    t = jnp.maximum(gsum[...] + _dot(e[...], w1c[...]) + b1[...], 0.0)
    t = jnp.maximum(_dot(t, w2[...]) + b2[...], 0.0)
    t = _dot(t, w3[...]) + b3[...]
    en = _ln(t, g[...], b[...])
    enew_out[...] = en
    eout_out[...] = e[...] + en


def _edge_step(gsum, e, w1c, b1, w2, b2, w3, b3, g, b):
    sds = jax.ShapeDtypeStruct((_EP, _L), jnp.float32)
    return pl.pallas_call(
        _edge_step_body,
        grid=(_EP // _BE,),
        in_specs=[_rows(_BE, _L)] * 2
                 + [_full((_L, _L)), _full((1, _L))] * 3
                 + [_full((1, _L)), _full((1, _L))],
        out_specs=[_rows(_BE, _L)] * 2,
        out_shape=[sds, sds],
    )(gsum, e, w1c, b1, w2, b2, w3, b3, g, b)


def _node_step_body(a0, a1, h, va, c1, vb, v2, c2, v3, c3, g, b, wa, wb,
                    h_out, ha_out, hb_out):
    a = a0[...] + a1[...]
    t = jnp.maximum(_dot(a, va[...]) + _dot(h[...], vb[...]) + c1[...], 0.0)
    t = jnp.maximum(_dot(t, v2[...]) + c2[...], 0.0)
    t = _dot(t, v3[...]) + c3[...]
    hn = _ln(t, g[...], b[...])
    ho = h[...] + hn
    h_out[...] = ho
    ha_out[...] = _dot(ho, wa[...])
    hb_out[...] = _dot(ho, wb[...])


def _node_step(a0, a1, h, va, c1, vb, v2, c2, v3, c3, g, b, wa, wb):
    sds = jax.ShapeDtypeStruct((_N, _L), jnp.float32)
    return pl.pallas_call(
        _node_step_body,
        grid=(_N // _BN,),
        in_specs=[_rows(_BN, _L)] * 3
                 + [_full((_L, _L)), _full((1, _L)), _full((_L, _L)),
                    _full((_L, _L)), _full((1, _L)),
                    _full((_L, _L)), _full((1, _L))]
                 + [_full((1, _L)), _full((1, _L)),
                    _full((_L, _L)), _full((_L, _L))],
        out_specs=[_rows(_BN, _L)] * 3,
        out_shape=[sds, sds, sds],
    )(a0, a1, h, va, c1, vb, v2, c2, v3, c3, g, b, wa, wb)


def _dec_body(h, w0, b0, w1, b1, w2, b2, out):
    z = jnp.maximum(_dot(h[...], w0[...]) + b0[...], 0.0)
    z = jnp.maximum(_dot(z, w1[...]) + b1[...], 0.0)
    out[...] = _dot(z, w2[...]) + b2[...]


def _dec(h, w0, b0, w1, b1, w2p, b2p):
    return pl.pallas_call(
        _dec_body,
        grid=(_N // _BN,),
        in_specs=[_rows(_BN, _L)] + [_full((_L, _L)), _full((1, _L))] * 3,
        out_specs=_rows(_BN, _L),
        out_shape=jax.ShapeDtypeStruct((_N, _L), jnp.float32),
    )(h, w0, b0, w1, b1, w2p, b2p)


_NB = 5


def _gather_body(ha, hb, dst3, src3, g, idxd, idxs,
                 bufa0, bufa1, bufa2, bufa3, bufa4,
                 bufb0, bufb1, bufb2, bufb3, bufb4,
                 sga0, sga1, sga2, sga3, sga4,
                 sgb0, sgb1, sgb2, sgb3, sgb4):
    wid = lax.axis_index("s") * _NC + lax.axis_index("c")
    cbase = wid * _CPT
    bufa = (bufa0, bufa1, bufa2, bufa3, bufa4)
    bufb = (bufb0, bufb1, bufb2, bufb3, bufb4)
    sga = (sga0, sga1, sga2, sga3, sga4)
    sgb = (sgb0, sgb1, sgb2, sgb3, sgb4)

    pltpu.sync_copy(dst3.at[wid], idxd)
    pltpu.sync_copy(src3.at[wid], idxs)

    for b in range(_NB):
        pltpu.async_copy(ha.at[idxd.at[b]], bufa[b], sga[b])
        pltpu.async_copy(hb.at[idxs.at[b]], bufb[b], sgb[b])

    def drain(b, ci):
        grow = (cbase + ci) * _CHUNK
        pltpu.make_async_copy(ha.at[idxd.at[0]], bufa[b], sga[b]).wait()
        pltpu.make_async_copy(hb.at[idxs.at[0]], bufb[b], sgb[b]).wait()

        def addrow(r, carry):
            for c in range(_L // 16):
                sl = pl.ds(c * 16, 16)
                bufa[b][r, sl] = bufa[b][r, sl] + bufb[b][r, sl]
            return carry

        lax.fori_loop(0, _CHUNK, addrow, 0)
        pltpu.sync_copy(bufa[b], g.at[pl.ds(grow, _CHUNK)])

    def ring(j, carry):
        for b in range(_NB):
            ci = j * _NB + b
            drain(b, ci)
            pltpu.async_copy(ha.at[idxd.at[ci + _NB]], bufa[b], sga[b])
            pltpu.async_copy(hb.at[idxs.at[ci + _NB]], bufb[b], sgb[b])
        return carry

    lax.fori_loop(0, _CPT // _NB - 1, ring, 0)
    for b in range(_NB):
        drain(b, _CPT - _NB + b)


def _scatter_body(enew, dst3, zeros, agg, idxa,
                  buf0, buf1, buf2, buf3, buf4,
                  sl0, sl1, sl2, sl3, sl4, acc):
    cid = lax.axis_index("c")
    sid = lax.axis_index("s")
    wid = sid * _NC + cid
    cbase = wid * _CPT
    bufs = (buf0, buf1, buf2, buf3, buf4)
    sl = (sl0, sl1, sl2, sl3, sl4)

    pltpu.sync_copy(zeros.at[pl.ds(sid * _RPT, _RPT)],
                    acc.at[pl.ds(sid * _RPT, _RPT)])
    pltpu.sync_copy(dst3.at[wid], idxa)
    plsc.subcore_barrier()

    for b in range(_NB):
        pltpu.async_copy(enew.at[pl.ds((cbase + b) * _CHUNK, _CHUNK)],
                         bufs[b], sl[b])

    def add(b, ci):
        pltpu.make_async_copy(enew.at[pl.ds(0, _CHUNK)], bufs[b],
                              sl[b]).wait()
        pltpu.sync_copy(bufs[b], acc.at[idxa.at[ci]], add=True)

    def pair(j, carry):
        for b in range(_NB):
            ci = j * _NB + b
            add(b, ci)
            pltpu.async_copy(
                enew.at[pl.ds((cbase + ci + _NB) * _CHUNK, _CHUNK)],
                bufs[b], sl[b])
        return carry

    lax.fori_loop(0, _CPT // _NB - 1, pair, 0)
    for b in range(_NB):
        add(b, _CPT - _NB + b)
    plsc.subcore_barrier()

    pltpu.sync_copy(acc.at[pl.ds(sid * _RPT, _RPT)],
                    agg.at[cid, pl.ds(sid * _RPT, _RPT)])


_SC_CALLS = {}


def _sc_calls():
    if not _SC_CALLS:
        mesh = plsc.VectorSubcoreMesh(core_axis_name="c", subcore_axis_name="s",
                                      num_cores=_NC, num_subcores=_NS)
        _SC_CALLS['gather'] = pl.kernel(
            _gather_body,
            out_type=jax.ShapeDtypeStruct((_EP, _L), jnp.float32),
            mesh=mesh,
            scratch_types=(
                [pltpu.VMEM((_CPT, _CHUNK), jnp.int32)] * 2
                + [pltpu.VMEM((_CHUNK, _L), jnp.float32)] * (2 * _NB)
                + [pltpu.SemaphoreType.DMA] * (2 * _NB)
            ),
        )
        _SC_CALLS['scatter'] = pl.kernel(
            _scatter_body,
            out_type=jax.ShapeDtypeStruct((_NC, _ROWS, _L), jnp.float32),
            mesh=mesh,
            scratch_types=(
                [pltpu.VMEM((_CPT, _CHUNK), jnp.int32)]
                + [pltpu.VMEM((_CHUNK, _L), jnp.float32)] * _NB
                + [pltpu.SemaphoreType.DMA] * _NB
                + [pltpu.VMEM_SHARED((_ROWS, _L), jnp.float32)]
            ),
        )
    return _SC_CALLS


def _sc_gather(ha, hb, dst_g, src_g):
    return _sc_calls()['gather'](ha, hb, dst_g, src_g)


def _sc_scatter(enew, dst_s, zeros):
    return _sc_calls()['scatter'](enew, dst_s, zeros)


def kernel(x, edge_index, edge_features, params):
    src = edge_index[0].astype(jnp.int32)
    dst = edge_index[1].astype(jnp.int32)
    dst_g = dst.reshape(_NW, _CPT, _CHUNK)
    src_g = src.reshape(_NW, _CPT, _CHUNK)
    efp = edge_features
    zeros = jnp.zeros((_ROWS, _L), jnp.float32)

    def r1(v):
        return v.reshape(1, -1)

    (enc_n_mlp, enc_n_ln) = params['enc_node']
    (enc_e_mlp, enc_e_ln) = params['enc_edge']
    inets = params['inets']

    w1a = [p['edge_mlp'][0][0][:_L] for p in inets]
    w1b = [p['edge_mlp'][0][0][_L:2 * _L] for p in inets]
    w1c = [p['edge_mlp'][0][0][2 * _L:] for p in inets]
    va = [p['node_mlp'][0][0][:_L] for p in inets]
    vb = [p['node_mlp'][0][0][_L:] for p in inets]

    e = _edge_enc(efp,
                  enc_e_mlp[0][0], r1(enc_e_mlp[0][1]),
                  enc_e_mlp[1][0], r1(enc_e_mlp[1][1]),
                  enc_e_mlp[2][0], r1(enc_e_mlp[2][1]),
                  r1(enc_e_ln[0]), r1(enc_e_ln[1]))
    h, ha, hb = _node_enc(x,
                          enc_n_mlp[0][0], r1(enc_n_mlp[0][1]),
                          enc_n_mlp[1][0], r1(enc_n_mlp[1][1]),
                          enc_n_mlp[2][0], r1(enc_n_mlp[2][1]),
                          r1(enc_n_ln[0]), r1(enc_n_ln[1]),
                          w1a[0], w1b[0])

    for s in range(_STEPS):
        p = inets[s]
        g = _sc_gather(ha, hb, dst_g, src_g)
        enew, e = _edge_step(g, e,
                             w1c[s], r1(p['edge_mlp'][0][1]),
                             p['edge_mlp'][1][0], r1(p['edge_mlp'][1][1]),
                             p['edge_mlp'][2][0], r1(p['edge_mlp'][2][1]),
                             r1(p['edge_ln'][0]), r1(p['edge_ln'][1]))
        aggp = _sc_scatter(enew, dst_g, zeros)
        nxt = (s + 1) % _STEPS
        h, ha, hb = _node_step(aggp[0, :_N], aggp[1, :_N], h,
                               va[s], r1(p['node_mlp'][0][1]),
                               vb[s],
                               p['node_mlp'][1][0], r1(p['node_mlp'][1][1]),
                               p['node_mlp'][2][0], r1(p['node_mlp'][2][1]),
                               r1(p['node_ln'][0]), r1(p['node_ln'][1]),
                               w1a[nxt], w1b[nxt])

    dec = params['dec']
    w2p = jnp.zeros((_L, _L), jnp.float32).at[:, :3].set(dec[2][0])
    b2p = jnp.zeros((1, _L), jnp.float32).at[0, :3].set(dec[2][1])
    y = _dec(h, dec[0][0], r1(dec[0][1]), dec[1][0], r1(dec[1][1]), w2p, b2p)
    return y[:, :3]


# speedup vs baseline: 1.1164x; 1.1164x over previous
import jax
import jax.numpy as jnp
from jax import lax
from jax.experimental import pallas as pl
from jax.experimental.pallas import tpu as pltpu
from jax.experimental.pallas import tpu_sc as plsc

_N = 10000
_E = 160000
_L = 128
_DE = 16
_STEPS = 5

_NC, _NS = 2, 16
_NW = _NC * _NS
_CHUNK = 40
_CPT = 125
_EP = _NW * _CPT * _CHUNK
_ROWS = 10112
_RPT = _ROWS // _NS

_CPT2 = 128
_EPT2 = _CPT2 * _CHUNK
_NB2 = 4

_BE = 1000
_BN = 1000


def _dot(a, b):
    return jnp.dot(a, b, preferred_element_type=jnp.float32)


def _ln(z, g, b):
    mu = jnp.mean(z, axis=-1, keepdims=True)
    var = jnp.mean((z - mu) ** 2, axis=-1, keepdims=True)
    return (z - mu) / jnp.sqrt(var + 1e-5) * g + b


def _full(shape):
    return pl.BlockSpec(shape, lambda i: (0,) * len(shape))


def _rows(bs, ncols):
    return pl.BlockSpec((bs, ncols), lambda i: (i, 0))


def _edge_enc_body(ef, w0, b0, w1, b1, w2, b2, out):
    z = jnp.maximum(_dot(ef[...], w0[...]) + b0[...], 0.0)
    z = jnp.maximum(_dot(z, w1[...]) + b1[...], 0.0)
    out[...] = _dot(z, w2[...]) + b2[...]


def _edge_enc(efp, w0, b0, w1, b1, w2, b2):
    return pl.pallas_call(
        _edge_enc_body,
        grid=(_EP // _BE,),
        in_specs=[_rows(_BE, _DE), _full((_DE, _L)), _full((1, _L)),
                  _full((_L, _L)), _full((1, _L)), _full((_L, _L)),
                  _full((1, _L))],
        out_specs=_rows(_BE, _L),
        out_shape=jax.ShapeDtypeStruct((_EP, _L), jnp.float32),
    )(efp, w0, b0, w1, b1, w2, b2)


def _node_enc_body(x, w0, b0, w1, b1, w2, b2, h_out):
    z = jnp.maximum(_dot(x[...], w0[...]) + b0[...], 0.0)
    z = jnp.maximum(_dot(z, w1[...]) + b1[...], 0.0)
    h_out[...] = _dot(z, w2[...]) + b2[...]


def _node_enc(x, w0, b0, w1, b1, w2, b2):
    return pl.pallas_call(
        _node_enc_body,
        grid=(_N // _BN,),
        in_specs=[_rows(_BN, _L)] + [_full((_L, _L)), _full((1, _L))] * 3,
        out_specs=_rows(_BN, _L),
        out_shape=jax.ShapeDtypeStruct((_N, _L), jnp.float32),
    )(x, w0, b0, w1, b1, w2, b2)


def _edge_step_body(gd, gs, e, w1, b1, w2, b2, w3, b3, t_out):
    cat = jnp.concatenate([gd[...], gs[...], e[...]], axis=-1)
    t = jnp.maximum(_dot(cat, w1[...]) + b1[...], 0.0)
    t = jnp.maximum(_dot(t, w2[...]) + b2[...], 0.0)
    t_out[...] = _dot(t, w3[...]) + b3[...]


def _edge_step(gd, gs, e, w1, b1, w2, b2, w3, b3):
    return pl.pallas_call(
        _edge_step_body,
        grid=(_EP // _BE,),
        in_specs=[_rows(_BE, _L)] * 3
                 + [_full((3 * _L, _L)), _full((1, _L)),
                    _full((_L, _L)), _full((1, _L)),
                    _full((_L, _L)), _full((1, _L))],
        out_specs=_rows(_BE, _L),
        out_shape=jax.ShapeDtypeStruct((_EP, _L), jnp.float32),
    )(gd, gs, e, w1, b1, w2, b2, w3, b3)


def _node_step_body(a0, a1, h, v1, c1, v2, c2, v3, c3, t_out):
    a = a0[...] + a1[...]
    t = jnp.concatenate([a, h[...]], axis=-1)
    t = jnp.maximum(_dot(t, v1[...]) + c1[...], 0.0)
    t = jnp.maximum(_dot(t, v2[...]) + c2[...], 0.0)
    t_out[...] = _dot(t, v3[...]) + c3[...]


def _node_step(a0, a1, h, v1, c1, v2, c2, v3, c3):
    return pl.pallas_call(
        _node_step_body,
        grid=(_N // _BN,),
        in_specs=[_rows(_BN, _L)] * 3
                 + [_full((2 * _L, _L)), _full((1, _L)),
                    _full((_L, _L)), _full((1, _L)),
                    _full((_L, _L)), _full((1, _L))],
        out_specs=_rows(_BN, _L),
        out_shape=jax.ShapeDtypeStruct((_N, _L), jnp.float32),
    )(a0, a1, h, v1, c1, v2, c2, v3, c3)


def _dec_body(h, w0, b0, w1, b1, w2, b2, out):
    z = jnp.maximum(_dot(h[...], w0[...]) + b0[...], 0.0)
    z = jnp.maximum(_dot(z, w1[...]) + b1[...], 0.0)
    out[...] = _dot(z, w2[...]) + b2[...]


def _dec(h, w0, b0, w1, b1, w2p, b2p):
    return pl.pallas_call(
        _dec_body,
        grid=(_N // _BN,),
        in_specs=[_rows(_BN, _L)] + [_full((_L, _L)), _full((1, _L))] * 3,
        out_specs=_rows(_BN, _L),
        out_shape=jax.ShapeDtypeStruct((_N, _L), jnp.float32),
    )(h, w0, b0, w1, b1, w2p, b2p)


_NB = 5


def _gather_body(h, dst3, src3, gd, gs, idxd, idxs,
                 bufa0, bufa1, bufa2, bufa3, bufa4,
                 bufb0, bufb1, bufb2, bufb3, bufb4,
                 sga0, sga1, sga2, sga3, sga4,
                 sgb0, sgb1, sgb2, sgb3, sgb4):
    wid = lax.axis_index("s") * _NC + lax.axis_index("c")
    cbase = wid * _CPT
    bufa = (bufa0, bufa1, bufa2, bufa3, bufa4)
    bufb = (bufb0, bufb1, bufb2, bufb3, bufb4)
    sga = (sga0, sga1, sga2, sga3, sga4)
    sgb = (sgb0, sgb1, sgb2, sgb3, sgb4)

    pltpu.sync_copy(dst3.at[wid], idxd)
    pltpu.sync_copy(src3.at[wid], idxs)

    for b in range(_NB):
        pltpu.async_copy(h.at[idxd.at[b]], bufa[b], sga[b])
        pltpu.async_copy(h.at[idxs.at[b]], bufb[b], sgb[b])

    def drain(b, ci):
        grow = (cbase + ci) * _CHUNK
        pltpu.make_async_copy(h.at[idxd.at[0]], bufa[b], sga[b]).wait()
        pltpu.make_async_copy(h.at[idxs.at[0]], bufb[b], sgb[b]).wait()
        pltpu.sync_copy(bufa[b], gd.at[pl.ds(grow, _CHUNK)])
        pltpu.sync_copy(bufb[b], gs.at[pl.ds(grow, _CHUNK)])

    def ring(j, carry):
        for b in range(_NB):
            ci = j * _NB + b
            drain(b, ci)
            pltpu.async_copy(h.at[idxd.at[ci + _NB]], bufa[b], sga[b])
            pltpu.async_copy(h.at[idxs.at[ci + _NB]], bufb[b], sgb[b])
        return carry

    lax.fori_loop(0, _CPT // _NB - 1, ring, 0)
    for b in range(_NB):
        drain(b, _CPT - _NB + b)


def _scatter_body(enew, perm1, dst3, zeros, agg, idxp, idxa,
                  buf0, buf1, buf2, buf3,
                  sl0, sl1, sl2, sl3, acc):
    cid = lax.axis_index("c")
    sid = lax.axis_index("s")
    wid = cid * _NS + sid
    bufs = (buf0, buf1, buf2, buf3)
    sl = (sl0, sl1, sl2, sl3)

    pltpu.sync_copy(zeros.at[pl.ds(sid * _RPT, _RPT)],
                    acc.at[pl.ds(sid * _RPT, _RPT)])
    pltpu.sync_copy(perm1.at[pl.ds(wid * _EPT2, _EPT2)], idxp)
    pltpu.sync_copy(dst3.at[wid], idxa)
    plsc.subcore_barrier()

    def pslice(ci):
        return idxp.at[pl.ds(ci * _CHUNK, _CHUNK)]

    for b in range(_NB2):
        pltpu.async_copy(enew.at[pslice(b)], bufs[b], sl[b])

    def add(b, ci):
        pltpu.make_async_copy(enew.at[pslice(0)], bufs[b], sl[b]).wait()
        pltpu.sync_copy(bufs[b], acc.at[idxa.at[ci]], add=True)

    def ring(j, carry):
        for b in range(_NB2):
            ci = j * _NB2 + b
            add(b, ci)
            pltpu.async_copy(enew.at[pslice(ci + _NB2)], bufs[b], sl[b])
        return carry

    lax.fori_loop(0, _CPT2 // _NB2 - 1, ring, 0)
    for b in range(_NB2):
        add(b, _CPT2 - _NB2 + b)
    plsc.subcore_barrier()

    pltpu.sync_copy(acc.at[pl.ds(sid * _RPT, _RPT)],
                    agg.at[cid, pl.ds(sid * _RPT, _RPT)])


_SC_CALLS = {}


def _sc_calls():
    if not _SC_CALLS:
        mesh = plsc.VectorSubcoreMesh(core_axis_name="c", subcore_axis_name="s",
                                      num_cores=_NC, num_subcores=_NS)
        _SC_CALLS['gather'] = pl.kernel(
            _gather_body,
            out_type=[jax.ShapeDtypeStruct((_EP, _L), jnp.float32)] * 2,
            mesh=mesh,
            scratch_types=(
                [pltpu.VMEM((_CPT, _CHUNK), jnp.int32)] * 2
                + [pltpu.VMEM((_CHUNK, _L), jnp.float32)] * (2 * _NB)
                + [pltpu.SemaphoreType.DMA] * (2 * _NB)
            ),
        )
        _SC_CALLS['scatter'] = pl.kernel(
            _scatter_body,
            out_type=jax.ShapeDtypeStruct((_NC, _ROWS, _L), jnp.float32),
            mesh=mesh,
            scratch_types=(
                [pltpu.VMEM((_EPT2,), jnp.int32)]
                + [pltpu.VMEM((_CPT2, _CHUNK), jnp.int32)]
                + [pltpu.VMEM((_CHUNK, _L), jnp.float32)] * _NB2
                + [pltpu.SemaphoreType.DMA] * _NB2
                + [pltpu.VMEM_SHARED((_ROWS, _L), jnp.float32)]
            ),
        )
    return _SC_CALLS


def _sc_gather(h, dst_g, src_g):
    return _sc_calls()['gather'](h, dst_g, src_g)


def _sc_scatter(enew, perm3, dst3, zeros):
    return _sc_calls()['scatter'](enew, perm3, dst3, zeros)


def kernel(x, edge_index, edge_features, params):
    src = edge_index[0].astype(jnp.int32)
    dst = edge_index[1].astype(jnp.int32)
    dst_g = dst.reshape(_NW, _CPT, _CHUNK)
    src_g = src.reshape(_NW, _CPT, _CHUNK)
    perm = jnp.argsort(dst, stable=True).astype(jnp.int32)
    dsts = dst[perm]
    bnodes = dsts[jnp.arange(_NW) * (_E // _NW)]
    split = jnp.searchsorted(dsts, bnodes).astype(jnp.int32)
    split = jnp.concatenate([split, jnp.array([_E], jnp.int32)])
    base = split[:-1, None] + jnp.arange(_EPT2, dtype=jnp.int32)[None, :]
    valid = base < split[1:, None]
    basec = jnp.minimum(base, _E - 1)
    perm_pad = jnp.where(valid, perm[basec], 0)
    dst_pad = jnp.where(valid, dsts[basec], _N)

    def _stride(a):
        return a.reshape(_NW, _CHUNK, _CPT2).transpose(0, 2, 1)

    perm_s = _stride(perm_pad).reshape(-1)
    dsts3 = _stride(dst_pad)
    efp = edge_features
    zeros = jnp.zeros((_ROWS, _L), jnp.float32)

    def r1(v):
        return v.reshape(1, -1)

    (enc_n_mlp, enc_n_ln) = params['enc_node']
    (enc_e_mlp, enc_e_ln) = params['enc_edge']
    inets = params['inets']

    e = _ln(_edge_enc(efp,
                      enc_e_mlp[0][0], r1(enc_e_mlp[0][1]),
                      enc_e_mlp[1][0], r1(enc_e_mlp[1][1]),
                      enc_e_mlp[2][0], r1(enc_e_mlp[2][1])),
            enc_e_ln[0], enc_e_ln[1])
    h = _ln(_node_enc(x,
                      enc_n_mlp[0][0], r1(enc_n_mlp[0][1]),
                      enc_n_mlp[1][0], r1(enc_n_mlp[1][1]),
                      enc_n_mlp[2][0], r1(enc_n_mlp[2][1])),
            enc_n_ln[0], enc_n_ln[1])

    for s in range(_STEPS):
        p = inets[s]
        gd, gs = _sc_gather(h, dst_g, src_g)
        t3 = _edge_step(gd, gs, e,
                        p['edge_mlp'][0][0], r1(p['edge_mlp'][0][1]),
                        p['edge_mlp'][1][0], r1(p['edge_mlp'][1][1]),
                        p['edge_mlp'][2][0], r1(p['edge_mlp'][2][1]))
        enew = _ln(t3, p['edge_ln'][0], p['edge_ln'][1])
        e = e + enew
        aggp = _sc_scatter(enew, perm_s, dsts3, zeros)
        u3 = _node_step(aggp[0, :_N], aggp[1, :_N], h,
                        p['node_mlp'][0][0], r1(p['node_mlp'][0][1]),
                        p['node_mlp'][1][0], r1(p['node_mlp'][1][1]),
                        p['node_mlp'][2][0], r1(p['node_mlp'][2][1]))
        h = h + _ln(u3, p['node_ln'][0], p['node_ln'][1])

    dec = params['dec']
    w2p = jnp.zeros((_L, _L), jnp.float32).at[:, :3].set(dec[2][0])
    b2p = jnp.zeros((1, _L), jnp.float32).at[0, :3].set(dec[2][1])
    y = _dec(h, dec[0][0], r1(dec[0][1]), dec[1][0], r1(dec[1][1]), w2p, b2p)
    return y[:, :3]
